# Initial kernel scaffold; baseline (speedup 1.0000x reference)
import jax
import jax.numpy as jnp
from jax.experimental import pallas as pl


def kernel(x, edge_index, W1, b1, W2, b2, W3, b3, W4, b4, gamma, beta):
    raise NotImplementedError("write your pallas kernel here")


# speedup vs baseline: 5.0162x; 5.0162x over previous
import functools

import jax
import jax.numpy as jnp
from jax import lax
from jax.experimental import pallas as pl
from jax.experimental.pallas import tpu as pltpu
from jax.experimental.pallas import tpu_sc as plsc

NC = 2
NS = 16
LANES = 16


def _premix_body(x_ref, wa_ref, wb_ref, b1_ref, a_ref, b_ref):
    xb = x_ref[...]
    a_ref[...] = jnp.dot(xb, wa_ref[...], preferred_element_type=jnp.float32)
    b_ref[...] = (
        jnp.dot(xb, wb_ref[...], preferred_element_type=jnp.float32)
        + b1_ref[...]
    )


def _premix(x, w1a, w1b, b1):
    n, d = x.shape
    bm = 2000
    return pl.pallas_call(
        _premix_body,
        grid=(n // bm,),
        in_specs=[
            pl.BlockSpec((bm, d), lambda i: (i, 0)),
            pl.BlockSpec((d, d), lambda i: (0, 0)),
            pl.BlockSpec((d, d), lambda i: (0, 0)),
            pl.BlockSpec((1, d), lambda i: (0, 0)),
        ],
        out_specs=[
            pl.BlockSpec((bm, d), lambda i: (i, 0)),
            pl.BlockSpec((bm, d), lambda i: (i, 0)),
        ],
        out_shape=[
            jax.ShapeDtypeStruct((n, d), jnp.float32),
            jax.ShapeDtypeStruct((n, d), jnp.float32),
        ],
    )(x, w1a, w1b, b1.reshape(1, d))


def _edge_agg(a, b, row, col):
    n, d = a.shape
    e = row.shape[0]
    nworkers = NC * NS
    ep = e // nworkers
    ch = 80
    nchunk = ep // ch
    br = 400
    nblk = n // br
    nround = -(-nblk // NS)
    dsl = d // LANES

    mesh = plsc.VectorSubcoreMesh(
        core_axis_name="c", subcore_axis_name="s",
        num_cores=NC, num_subcores=NS)

    @functools.partial(
        pl.kernel,
        out_type=[
            jax.ShapeDtypeStruct((NC * n, d), jnp.float32),
            jax.ShapeDtypeStruct((nworkers * n,), jnp.float32),
        ],
        mesh=mesh,
        scratch_types=[
            pltpu.VMEM((ch,), jnp.int32),
            pltpu.VMEM((ch,), jnp.int32),
            pltpu.VMEM((ch, d), jnp.float32),
            pltpu.VMEM((ch, d), jnp.float32),
            pltpu.VMEM((n,), jnp.float32),
            pltpu.VMEM_SHARED((n, d), jnp.float32),
            pltpu.SemaphoreType.DMA,
            pltpu.SemaphoreType.DMA,
        ],
        compiler_params=pltpu.CompilerParams(use_tc_tiling_on_sc=False,
                                             needs_layout_passes=False),
    )
    def k(a_hbm, b_hbm, row_hbm, col_hbm, agg_out, deg_out,
          idx_row, idx_col, abuf, bbuf, deg_v, agg_sh, sem_a, sem_b):
        ci = lax.axis_index("c")
        si = lax.axis_index("s")
        wid = si * NC + ci

        zeros16 = jnp.zeros((LANES,), jnp.float32)
        ones16 = jnp.ones((LANES,), jnp.float32)

        def zrow_body(r, _):
            for j in range(dsl):
                abuf[r, pl.ds(j * LANES, LANES)] = zeros16
            return 0
        lax.fori_loop(0, ch, zrow_body, 0)

        def dz_body(r, _):
            deg_v[pl.ds(r * LANES, LANES)] = zeros16
            return 0
        lax.fori_loop(0, n // LANES, dz_body, 0)

        for rnd in range(nround):
            bi = si + NS * rnd

            @pl.when(bi < nblk)
            def _():
                for z in range(br // ch):
                    pltpu.sync_copy(
                        abuf, agg_sh.at[pl.ds(bi * br + z * ch, ch)])
        plsc.subcore_barrier()

        def chunk_body(t, _):
            base = wid * ep + t * ch
            pltpu.sync_copy(row_hbm.at[pl.ds(base, ch)], idx_row)
            pltpu.sync_copy(col_hbm.at[pl.ds(base, ch)], idx_col)
            cp_a = pltpu.async_copy(a_hbm.at[idx_row], abuf, sem_a)
            cp_b = pltpu.async_copy(b_hbm.at[idx_col], bbuf, sem_b)
            cp_a.wait()
            cp_b.wait()

            def e_body(ei, _):
                for j in range(dsl):
                    sl = pl.ds(j * LANES, LANES)
                    bbuf[ei, sl] = jnp.maximum(abuf[ei, sl] + bbuf[ei, sl],
                                               0.0)
                return 0
            lax.fori_loop(0, ch, e_body, 0)

            pltpu.sync_copy(bbuf, agg_sh.at[idx_col], add=True)
            for kk in range(ch // LANES):
                plsc.addupdate_scatter(
                    deg_v, [idx_col[pl.ds(kk * LANES, LANES)]], ones16)
            return 0
        lax.fori_loop(0, nchunk, chunk_body, 0)

        plsc.subcore_barrier()
        for rnd in range(nround):
            bi = si + NS * rnd

            @pl.when(bi < nblk)
            def _():
                pltpu.sync_copy(agg_sh.at[pl.ds(bi * br, br)],
                                agg_out.at[pl.ds(ci * n + bi * br, br)])
        pltpu.sync_copy(deg_v, deg_out.at[pl.ds(wid * n, n)])

    return k(a, b, row, col)


def _update_body(x_ref, g0, g1, dgr, w2, b2r, w3a, w3b, b3r, w4, b4r,
                 gm, bt, o_ref):
    xb = x_ref[...]
    agg = jnp.dot(g0[...] + g1[...], w2[...],
                  preferred_element_type=jnp.float32)
    deg = jnp.sum(dgr[...], axis=1, keepdims=True)
    agg = agg + deg * b2r[...]
    h = jnp.maximum(
        jnp.dot(xb, w3a[...], preferred_element_type=jnp.float32)
        + jnp.dot(agg, w3b[...], preferred_element_type=jnp.float32)
        + b3r[...], 0.0)
    u = jnp.dot(h, w4[...], preferred_element_type=jnp.float32) + b4r[...]
    mu = jnp.mean(u, axis=-1, keepdims=True)
    var = jnp.mean((u - mu) ** 2, axis=-1, keepdims=True)
    ln = (u - mu) * lax.rsqrt(var + 1e-5) * gm[...] + bt[...]
    o_ref[...] = ln + xb


def _update(x, g0, g1, degt, w2, b2, w3a, w3b, b3, w4, b4, gamma, beta):
    n, d = x.shape
    nw = degt.shape[1]
    bm = 2000
    full = lambda i: (0, 0)
    blk = lambda i: (i, 0)
    return pl.pallas_call(
        _update_body,
        grid=(n // bm,),
        in_specs=[
            pl.BlockSpec((bm, d), blk),
            pl.BlockSpec((bm, d), blk),
            pl.BlockSpec((bm, d), blk),
            pl.BlockSpec((bm, nw), blk),
            pl.BlockSpec((d, d), full),
            pl.BlockSpec((1, d), full),
            pl.BlockSpec((d, d), full),
            pl.BlockSpec((d, d), full),
            pl.BlockSpec((1, d), full),
            pl.BlockSpec((d, d), full),
            pl.BlockSpec((1, d), full),
            pl.BlockSpec((1, d), full),
            pl.BlockSpec((1, d), full),
        ],
        out_specs=pl.BlockSpec((bm, d), blk),
        out_shape=jax.ShapeDtypeStruct((n, d), jnp.float32),
    )(x, g0, g1, degt, w2, b2.reshape(1, d), w3a, w3b, b3.reshape(1, d),
      w4, b4.reshape(1, d), gamma.reshape(1, d), beta.reshape(1, d))


def kernel(x, edge_index, W1, b1, W2, b2, W3, b3, W4, b4, gamma, beta):
    n, d = x.shape
    row = edge_index[0]
    col = edge_index[1]
    a, bnode = _premix(x, W1[:d], W1[d:], b1)
    aggp, degp = _edge_agg(a, bnode, row, col)
    degt = degp.reshape(NC * NS, n).T
    return _update(x, aggp[:n], aggp[n:], degt,
                   W2, b2, W3[:d], W3[d:], b3, W4, b4, gamma, beta)


# speedup vs baseline: 6.4143x; 1.2787x over previous
import functools

import jax
import jax.numpy as jnp
from jax import lax
from jax.experimental import pallas as pl
from jax.experimental.pallas import tpu as pltpu
from jax.experimental.pallas import tpu_sc as plsc

NC = 2
NS = 16
LANES = 16


def _premix_body(x_ref, wa_ref, wb_ref, b1_ref, a_ref, b_ref):
    xb = x_ref[...]
    a_ref[...] = jnp.dot(xb, wa_ref[...], preferred_element_type=jnp.float32)
    b_ref[...] = (
        jnp.dot(xb, wb_ref[...], preferred_element_type=jnp.float32)
        + b1_ref[...]
    )


def _premix(x, w1a, w1b, b1):
    n, d = x.shape
    bm = 2000
    return pl.pallas_call(
        _premix_body,
        grid=(n // bm,),
        in_specs=[
            pl.BlockSpec((bm, d), lambda i: (i, 0)),
            pl.BlockSpec((d, d), lambda i: (0, 0)),
            pl.BlockSpec((d, d), lambda i: (0, 0)),
            pl.BlockSpec((1, d), lambda i: (0, 0)),
        ],
        out_specs=[
            pl.BlockSpec((bm, d), lambda i: (i, 0)),
            pl.BlockSpec((bm, d), lambda i: (i, 0)),
        ],
        out_shape=[
            jax.ShapeDtypeStruct((n, d), jnp.float32),
            jax.ShapeDtypeStruct((n, d), jnp.float32),
        ],
    )(x, w1a, w1b, b1.reshape(1, d))


def _edge_agg(a, b, idx):
    n, d = a.shape
    nworkers = NC * NS
    ch = idx.shape[2]
    nchunk = idx.shape[0] // nworkers
    br = 400
    nblk = n // br
    nround = -(-nblk // NS)
    dsl = d // LANES

    mesh = plsc.VectorSubcoreMesh(
        core_axis_name="c", subcore_axis_name="s",
        num_cores=NC, num_subcores=NS)

    @functools.partial(
        pl.kernel,
        out_type=[
            jax.ShapeDtypeStruct((NC * n, d), jnp.float32),
            jax.ShapeDtypeStruct((nworkers * n,), jnp.float32),
        ],
        mesh=mesh,
        scratch_types=[
            pltpu.VMEM((4, ch), jnp.int32),
            pltpu.VMEM((2, ch, d), jnp.float32),
            pltpu.VMEM((2, ch, d), jnp.float32),
            pltpu.VMEM((n,), jnp.float32),
            pltpu.VMEM_SHARED((n, d), jnp.float32),
            pltpu.SemaphoreType.DMA,
            pltpu.SemaphoreType.DMA,
            pltpu.SemaphoreType.DMA,
            pltpu.SemaphoreType.DMA,
        ],
        compiler_params=pltpu.CompilerParams(use_tc_tiling_on_sc=False,
                                             needs_layout_passes=False),
    )
    def k(a_hbm, b_hbm, idx_hbm, agg_out, deg_out,
          idx_v, abuf, bbuf, deg_v, agg_sh,
          sga0, sgb0, sga1, sgb1):
        ci = lax.axis_index("c")
        si = lax.axis_index("s")
        wid = si * NC + ci
        tbase = wid * nchunk
        sga = (sga0, sga1)
        sgb = (sgb0, sgb1)

        zeros16 = jnp.zeros((LANES,), jnp.float32)
        ones16 = jnp.ones((LANES,), jnp.float32)

        def fetch_and_gather(t_dyn, slot):
            pltpu.sync_copy(idx_hbm.at[tbase + t_dyn],
                            idx_v.at[pl.ds(2 * slot, 2)])
            pltpu.async_copy(a_hbm.at[idx_v.at[2 * slot]], abuf.at[slot],
                             sga[slot])
            pltpu.async_copy(b_hbm.at[idx_v.at[2 * slot + 1]], bbuf.at[slot],
                             sgb[slot])

        def wait_gather(slot):
            pltpu.make_async_copy(a_hbm.at[idx_v.at[2 * slot]], abuf.at[slot],
                                  sga[slot]).wait()
            pltpu.make_async_copy(b_hbm.at[idx_v.at[2 * slot + 1]],
                                  bbuf.at[slot], sgb[slot]).wait()

        def zrow_body(r, _):
            for j in range(dsl):
                abuf[0, r, pl.ds(j * LANES, LANES)] = zeros16
            return 0
        lax.fori_loop(0, ch, zrow_body, 0)

        def dz_body(r, _):
            deg_v[pl.ds(r * LANES, LANES)] = zeros16
            return 0
        lax.fori_loop(0, n // LANES, dz_body, 0)

        for rnd in range(nround):
            bi = si + NS * rnd

            @pl.when(bi < nblk)
            def _():
                for z in range(br // ch):
                    pltpu.sync_copy(
                        abuf.at[0], agg_sh.at[pl.ds(bi * br + z * ch, ch)])
        plsc.subcore_barrier()

        fetch_and_gather(0, 0)

        def pair_body(p, _):
            for s in (0, 1):
                t = 2 * p + s
                nxt = 1 - s

                @pl.when(t + 1 < nchunk)
                def _():
                    fetch_and_gather(t + 1, nxt)

                wait_gather(s)

                def e_body(ei, _):
                    for j in range(dsl):
                        sl = pl.ds(j * LANES, LANES)
                        bbuf[s, ei, sl] = jnp.maximum(
                            abuf[s, ei, sl] + bbuf[s, ei, sl], 0.0)
                    return 0
                lax.fori_loop(0, ch, e_body, 0)

                pltpu.sync_copy(bbuf.at[s], agg_sh.at[idx_v.at[2 * s + 1]],
                                add=True)
                for kk in range(ch // LANES):
                    plsc.addupdate_scatter(
                        deg_v, [idx_v[2 * s + 1, pl.ds(kk * LANES, LANES)]],
                        ones16)
                rem = ch - (ch // LANES) * LANES
                if rem:
                    tail = lax.iota(jnp.int32, LANES) >= (LANES - rem)
                    plsc.addupdate_scatter(
                        deg_v,
                        [idx_v[2 * s + 1, pl.ds(ch - LANES, LANES)]],
                        ones16, mask=tail)
            return 0
        lax.fori_loop(0, nchunk // 2, pair_body, 0)

        plsc.subcore_barrier()
        for rnd in range(nround):
            bi = si + NS * rnd

            @pl.when(bi < nblk)
            def _():
                pltpu.sync_copy(agg_sh.at[pl.ds(bi * br, br)],
                                agg_out.at[pl.ds(ci * n + bi * br, br)])
        pltpu.sync_copy(deg_v, deg_out.at[pl.ds(wid * n, n)])

    return k(a, b, idx)


def _update_body(x_ref, g0, g1, dgr, w2, b2r, w3a, w3b, b3r, w4, b4r,
                 gm, bt, o_ref):
    xb = x_ref[...]
    agg = jnp.dot(g0[...] + g1[...], w2[...],
                  preferred_element_type=jnp.float32)
    deg = jnp.sum(dgr[...], axis=1, keepdims=True)
    agg = agg + deg * b2r[...]
    h = jnp.maximum(
        jnp.dot(xb, w3a[...], preferred_element_type=jnp.float32)
        + jnp.dot(agg, w3b[...], preferred_element_type=jnp.float32)
        + b3r[...], 0.0)
    u = jnp.dot(h, w4[...], preferred_element_type=jnp.float32) + b4r[...]
    mu = jnp.mean(u, axis=-1, keepdims=True)
    var = jnp.mean((u - mu) ** 2, axis=-1, keepdims=True)
    ln = (u - mu) * lax.rsqrt(var + 1e-5) * gm[...] + bt[...]
    o_ref[...] = ln + xb


def _update(x, g0, g1, degt, w2, b2, w3a, w3b, b3, w4, b4, gamma, beta):
    n, d = x.shape
    nw = degt.shape[1]
    bm = 2000
    full = lambda i: (0, 0)
    blk = lambda i: (i, 0)
    return pl.pallas_call(
        _update_body,
        grid=(n // bm,),
        in_specs=[
            pl.BlockSpec((bm, d), blk),
            pl.BlockSpec((bm, d), blk),
            pl.BlockSpec((bm, d), blk),
            pl.BlockSpec((bm, nw), blk),
            pl.BlockSpec((d, d), full),
            pl.BlockSpec((1, d), full),
            pl.BlockSpec((d, d), full),
            pl.BlockSpec((d, d), full),
            pl.BlockSpec((1, d), full),
            pl.BlockSpec((d, d), full),
            pl.BlockSpec((1, d), full),
            pl.BlockSpec((1, d), full),
            pl.BlockSpec((1, d), full),
        ],
        out_specs=pl.BlockSpec((bm, d), blk),
        out_shape=jax.ShapeDtypeStruct((n, d), jnp.float32),
    )(x, g0, g1, degt, w2, b2.reshape(1, d), w3a, w3b, b3.reshape(1, d),
      w4, b4.reshape(1, d), gamma.reshape(1, d), beta.reshape(1, d))


def kernel(x, edge_index, W1, b1, W2, b2, W3, b3, W4, b4, gamma, beta):
    n, d = x.shape
    e = edge_index.shape[1]
    nw = NC * NS
    ch = 40
    nchunk = e // (nw * ch)
    idx = jnp.concatenate(
        [edge_index[0].reshape(nw, nchunk, 1, ch),
         edge_index[1].reshape(nw, nchunk, 1, ch)],
        axis=2).reshape(nw * nchunk, 2, ch)
    a, bnode = _premix(x, W1[:d], W1[d:], b1)
    aggp, degp = _edge_agg(a, bnode, idx)
    degt = degp.reshape(NC * NS, n).T
    return _update(x, aggp[:n], aggp[n:], degt,
                   W2, b2, W3[:d], W3[d:], b3, W4, b4, gamma, beta)


# speedup vs baseline: 7.0321x; 1.0963x over previous
import functools

import jax
import jax.numpy as jnp
from jax import lax
from jax.experimental import pallas as pl
from jax.experimental.pallas import tpu as pltpu
from jax.experimental.pallas import tpu_sc as plsc

NC = 2
NS = 16
LANES = 16


def _premix_body(x_ref, wa_ref, wb_ref, b1_ref, a_ref, b_ref):
    xb = x_ref[...]
    a_ref[...] = jnp.dot(xb, wa_ref[...], preferred_element_type=jnp.float32)
    b_ref[...] = (
        jnp.dot(xb, wb_ref[...], preferred_element_type=jnp.float32)
        + b1_ref[...]
    )


def _premix(x, w1a, w1b, b1):
    n, d = x.shape
    bm = 2000
    return pl.pallas_call(
        _premix_body,
        grid=(n // bm,),
        in_specs=[
            pl.BlockSpec((bm, d), lambda i: (i, 0)),
            pl.BlockSpec((d, d), lambda i: (0, 0)),
            pl.BlockSpec((d, d), lambda i: (0, 0)),
            pl.BlockSpec((1, d), lambda i: (0, 0)),
        ],
        out_specs=[
            pl.BlockSpec((bm, d), lambda i: (i, 0)),
            pl.BlockSpec((bm, d), lambda i: (i, 0)),
        ],
        out_shape=[
            jax.ShapeDtypeStruct((n, d), jnp.float32),
            jax.ShapeDtypeStruct((n, d), jnp.float32),
        ],
    )(x, w1a, w1b, b1.reshape(1, d))


def _edge_agg(a, b, idx):
    n, d = a.shape
    nworkers = NC * NS
    ch = idx.shape[2]
    K = idx.shape[1] // 2
    nsup = idx.shape[0] // nworkers
    nchunk = nsup * K
    br = 400
    nblk = n // br
    nround = -(-nblk // NS)
    dsl = d // LANES

    mesh = plsc.VectorSubcoreMesh(
        core_axis_name="c", subcore_axis_name="s",
        num_cores=NC, num_subcores=NS)

    @functools.partial(
        pl.kernel,
        out_type=[
            jax.ShapeDtypeStruct((NC * n, d), jnp.float32),
            jax.ShapeDtypeStruct((nworkers * n,), jnp.float32),
        ],
        mesh=mesh,
        scratch_types=[
            pltpu.VMEM((4 * K, ch), jnp.int32),
            pltpu.VMEM((2, ch, d), jnp.float32),
            pltpu.VMEM((2, ch, d), jnp.float32),
            pltpu.VMEM((n,), jnp.float32),
            pltpu.VMEM_SHARED((n, d), jnp.float32),
            pltpu.SemaphoreType.DMA,
            pltpu.SemaphoreType.DMA,
            pltpu.SemaphoreType.DMA,
            pltpu.SemaphoreType.DMA,
        ],
        compiler_params=pltpu.CompilerParams(use_tc_tiling_on_sc=False,
                                             needs_layout_passes=False),
    )
    def k(a_hbm, b_hbm, idx_hbm, agg_out, deg_out,
          idx_v, abuf, bbuf, deg_v, agg_sh,
          sga0, sgb0, sga1, sgb1):
        ci = lax.axis_index("c")
        si = lax.axis_index("s")
        wid = si * NC + ci
        tsup = wid * nsup
        sga = (sga0, sga1)
        sgb = (sgb0, sgb1)

        zeros16 = jnp.zeros((LANES,), jnp.float32)
        ones16 = jnp.ones((LANES,), jnp.float32)

        def gather_issue(rowbase, gslot):
            pltpu.async_copy(a_hbm.at[idx_v.at[rowbase]], abuf.at[gslot],
                             sga[gslot])
            pltpu.async_copy(b_hbm.at[idx_v.at[rowbase + 1]], bbuf.at[gslot],
                             sgb[gslot])

        def gather_wait(rowbase, gslot):
            pltpu.make_async_copy(a_hbm.at[idx_v.at[rowbase]],
                                  abuf.at[gslot], sga[gslot]).wait()
            pltpu.make_async_copy(b_hbm.at[idx_v.at[rowbase + 1]],
                                  bbuf.at[gslot], sgb[gslot]).wait()

        def zrow_body(r, _):
            for j in range(dsl):
                abuf[0, r, pl.ds(j * LANES, LANES)] = zeros16
            return 0
        lax.fori_loop(0, ch, zrow_body, 0)

        def dz_body(r, _):
            deg_v[pl.ds(r * LANES, LANES)] = zeros16
            return 0
        lax.fori_loop(0, n // LANES, dz_body, 0)

        for rnd in range(nround):
            bi = si + NS * rnd

            @pl.when(bi < nblk)
            def _():
                for z in range(br // ch):
                    pltpu.sync_copy(
                        abuf.at[0], agg_sh.at[pl.ds(bi * br + z * ch, ch)])
        plsc.subcore_barrier()

        def chunk_run(sup_dyn, u, j, prefetch_next, fetch_next_sup):
            base = 2 * K * u
            nxtbase = 2 * K * (1 - u)
            if fetch_next_sup and j == 0:
                pltpu.sync_copy(idx_hbm.at[tsup + sup_dyn + 1],
                                idx_v.at[pl.ds(nxtbase, 2 * K)])
            if prefetch_next:
                if j < K - 1:
                    gather_issue(base + 2 * (j + 1), 1 - (j % 2))
                else:
                    gather_issue(nxtbase, 1 - (j % 2))
            gs = j % 2
            rb = base + 2 * j
            gather_wait(rb, gs)

            def e_body(ei, _):
                for jj in range(dsl):
                    sl = pl.ds(jj * LANES, LANES)
                    bbuf[gs, ei, sl] = jnp.maximum(
                        abuf[gs, ei, sl] + bbuf[gs, ei, sl], 0.0)
                return 0
            lax.fori_loop(0, ch, e_body, 0)

            pltpu.sync_copy(bbuf.at[gs], agg_sh.at[idx_v.at[rb + 1]],
                            add=True)
            for kk in range(ch // LANES):
                plsc.addupdate_scatter(
                    deg_v, [idx_v[rb + 1, pl.ds(kk * LANES, LANES)]],
                    ones16)
            rem = ch - (ch // LANES) * LANES
            if rem:
                tail = lax.iota(jnp.int32, LANES) >= (LANES - rem)
                plsc.addupdate_scatter(
                    deg_v, [idx_v[rb + 1, pl.ds(ch - LANES, LANES)]],
                    ones16, mask=tail)

        pltpu.sync_copy(idx_hbm.at[tsup], idx_v.at[pl.ds(0, 2 * K)])
        gather_issue(0, 0)

        def sup_pair(q, _):
            for u in (0, 1):
                for j in range(K):
                    chunk_run(2 * q + u, u, j, True, True)
            return 0
        lax.fori_loop(0, (nsup - 1) // 2, sup_pair, 0)
        for j in range(K):
            chunk_run(nsup - 1, 0, j, j < K - 1, False)

        plsc.subcore_barrier()
        for rnd in range(nround):
            bi = si + NS * rnd

            @pl.when(bi < nblk)
            def _():
                pltpu.sync_copy(agg_sh.at[pl.ds(bi * br, br)],
                                agg_out.at[pl.ds(ci * n + bi * br, br)])
        pltpu.sync_copy(deg_v, deg_out.at[pl.ds(wid * n, n)])

    return k(a, b, idx)


def _update_body(x_ref, g0, g1, dgr, w2, b2r, w3a, w3b, b3r, w4, b4r,
                 gm, bt, o_ref):
    xb = x_ref[...]
    agg = jnp.dot(g0[...] + g1[...], w2[...],
                  preferred_element_type=jnp.float32)
    deg = jnp.sum(dgr[...], axis=1, keepdims=True)
    agg = agg + deg * b2r[...]
    h = jnp.maximum(
        jnp.dot(xb, w3a[...], preferred_element_type=jnp.float32)
        + jnp.dot(agg, w3b[...], preferred_element_type=jnp.float32)
        + b3r[...], 0.0)
    u = jnp.dot(h, w4[...], preferred_element_type=jnp.float32) + b4r[...]
    mu = jnp.mean(u, axis=-1, keepdims=True)
    var = jnp.mean((u - mu) ** 2, axis=-1, keepdims=True)
    ln = (u - mu) * lax.rsqrt(var + 1e-5) * gm[...] + bt[...]
    o_ref[...] = ln + xb


def _update(x, g0, g1, degt, w2, b2, w3a, w3b, b3, w4, b4, gamma, beta):
    n, d = x.shape
    nw = degt.shape[1]
    bm = 2000
    full = lambda i: (0, 0)
    blk = lambda i: (i, 0)
    return pl.pallas_call(
        _update_body,
        grid=(n // bm,),
        in_specs=[
            pl.BlockSpec((bm, d), blk),
            pl.BlockSpec((bm, d), blk),
            pl.BlockSpec((bm, d), blk),
            pl.BlockSpec((bm, nw), blk),
            pl.BlockSpec((d, d), full),
            pl.BlockSpec((1, d), full),
            pl.BlockSpec((d, d), full),
            pl.BlockSpec((d, d), full),
            pl.BlockSpec((1, d), full),
            pl.BlockSpec((d, d), full),
            pl.BlockSpec((1, d), full),
            pl.BlockSpec((1, d), full),
            pl.BlockSpec((1, d), full),
        ],
        out_specs=pl.BlockSpec((bm, d), blk),
        out_shape=jax.ShapeDtypeStruct((n, d), jnp.float32),
    )(x, g0, g1, degt, w2, b2.reshape(1, d), w3a, w3b, b3.reshape(1, d),
      w4, b4.reshape(1, d), gamma.reshape(1, d), beta.reshape(1, d))


def kernel(x, edge_index, W1, b1, W2, b2, W3, b3, W4, b4, gamma, beta):
    n, d = x.shape
    e = edge_index.shape[1]
    nw = NC * NS
    ch = 40
    kk = 10
    nchunk = e // (nw * ch)
    nsup = nchunk // kk
    assert nsup * kk == nchunk and nsup % 2 == 1
    idx = jnp.concatenate(
        [edge_index[0].reshape(nw, nsup, kk, 1, ch),
         edge_index[1].reshape(nw, nsup, kk, 1, ch)],
        axis=3).reshape(nw * nsup, 2 * kk, ch)
    a, bnode = _premix(x, W1[:d], W1[d:], b1)
    aggp, degp = _edge_agg(a, bnode, idx)
    degt = degp.reshape(NC * NS, n).T
    return _update(x, aggp[:n], aggp[n:], degt,
                   W2, b2, W3[:d], W3[d:], b3, W4, b4, gamma, beta)


# speedup vs baseline: 8.2712x; 1.1762x over previous
import functools

import jax
import jax.numpy as jnp
from jax import lax
from jax.experimental import pallas as pl
from jax.experimental.pallas import tpu as pltpu
from jax.experimental.pallas import tpu_sc as plsc

NC = 2
NS = 16
LANES = 16


def _premix_body(x_ref, wa_ref, wb_ref, b1_ref, a_ref, b_ref):
    xb = x_ref[...]
    a_ref[...] = jnp.dot(xb, wa_ref[...], preferred_element_type=jnp.float32)
    b_ref[...] = (
        jnp.dot(xb, wb_ref[...], preferred_element_type=jnp.float32)
        + b1_ref[...]
    )


def _premix(x, w1a, w1b, b1):
    n, d = x.shape
    bm = 2000
    return pl.pallas_call(
        _premix_body,
        grid=(n // bm,),
        in_specs=[
            pl.BlockSpec((bm, d), lambda i: (i, 0)),
            pl.BlockSpec((d, d), lambda i: (0, 0)),
            pl.BlockSpec((d, d), lambda i: (0, 0)),
            pl.BlockSpec((1, d), lambda i: (0, 0)),
        ],
        out_specs=[
            pl.BlockSpec((bm, d), lambda i: (i, 0)),
            pl.BlockSpec((bm, d), lambda i: (i, 0)),
        ],
        out_shape=[
            jax.ShapeDtypeStruct((n, d), jnp.float32),
            jax.ShapeDtypeStruct((n, d), jnp.float32),
        ],
    )(x, w1a, w1b, b1.reshape(1, d))


def _edge_agg(a, b, rowm, colm):
    n, d = a.shape
    nworkers = NC * NS
    ch = rowm.shape[2]
    K = rowm.shape[1]
    nsup = rowm.shape[0] // nworkers
    nchunk = nsup * K
    br = 400
    nblk = n // br
    nround = -(-nblk // NS)
    dsl = d // LANES

    mesh = plsc.VectorSubcoreMesh(
        core_axis_name="c", subcore_axis_name="s",
        num_cores=NC, num_subcores=NS)

    @functools.partial(
        pl.kernel,
        out_type=[
            jax.ShapeDtypeStruct((NC * n, d), jnp.float32),
            jax.ShapeDtypeStruct((nworkers * n,), jnp.float32),
        ],
        mesh=mesh,
        scratch_types=[
            pltpu.VMEM((4 * K, ch), jnp.int32),
            pltpu.VMEM((2, ch, d), jnp.float32),
            pltpu.VMEM((2, ch, d), jnp.float32),
            pltpu.VMEM((n,), jnp.float32),
            pltpu.VMEM_SHARED((n, d), jnp.float32),
            pltpu.SemaphoreType.DMA,
            pltpu.SemaphoreType.DMA,
            pltpu.SemaphoreType.DMA,
            pltpu.SemaphoreType.DMA,
        ],
        compiler_params=pltpu.CompilerParams(use_tc_tiling_on_sc=False,
                                             needs_layout_passes=False),
    )
    def k(a_hbm, b_hbm, row_hbm, col_hbm, agg_out, deg_out,
          idx_v, abuf, bbuf, deg_v, agg_sh,
          sga0, sgb0, sga1, sgb1):
        ci = lax.axis_index("c")
        si = lax.axis_index("s")
        wid = si * NC + ci
        tsup = wid * nsup
        sga = (sga0, sga1)
        sgb = (sgb0, sgb1)

        zeros16 = jnp.zeros((LANES,), jnp.float32)
        ones16 = jnp.ones((LANES,), jnp.float32)

        def gather_issue(rrow, rcol, gslot):
            pltpu.async_copy(a_hbm.at[idx_v.at[rrow]], abuf.at[gslot],
                             sga[gslot])
            pltpu.async_copy(b_hbm.at[idx_v.at[rcol]], bbuf.at[gslot],
                             sgb[gslot])

        def gather_wait(rrow, rcol, gslot):
            pltpu.make_async_copy(a_hbm.at[idx_v.at[rrow]],
                                  abuf.at[gslot], sga[gslot]).wait()
            pltpu.make_async_copy(b_hbm.at[idx_v.at[rcol]],
                                  bbuf.at[gslot], sgb[gslot]).wait()

        def sup_fetch(sup_dyn, slot):
            pltpu.sync_copy(row_hbm.at[tsup + sup_dyn],
                            idx_v.at[pl.ds(2 * K * slot, K)])
            pltpu.sync_copy(col_hbm.at[tsup + sup_dyn],
                            idx_v.at[pl.ds(2 * K * slot + K, K)])

        def zrow_body(r, _):
            for j in range(dsl):
                abuf[0, r, pl.ds(j * LANES, LANES)] = zeros16
            return 0
        lax.fori_loop(0, ch, zrow_body, 0)

        def dz_body(r, _):
            deg_v[pl.ds(r * LANES, LANES)] = zeros16
            return 0
        lax.fori_loop(0, n // LANES, dz_body, 0)

        for rnd in range(nround):
            bi = si + NS * rnd

            @pl.when(bi < nblk)
            def _():
                for z in range(br // ch):
                    pltpu.sync_copy(
                        abuf.at[0], agg_sh.at[pl.ds(bi * br + z * ch, ch)])
        plsc.subcore_barrier()

        def chunk_run(sup_dyn, u, j, prefetch_next, fetch_next_sup):
            base = 2 * K * u
            nxtbase = 2 * K * (1 - u)
            if fetch_next_sup and j == 0:
                sup_fetch(sup_dyn + 1, 1 - u)
            if prefetch_next:
                if j < K - 1:
                    gather_issue(base + j + 1, base + K + j + 1,
                                 1 - (j % 2))
                else:
                    gather_issue(nxtbase, nxtbase + K, 1 - (j % 2))
            gs = j % 2
            rb = base + K + j
            gather_wait(base + j, rb, gs)

            def e_body(ei, _):
                for jj in range(dsl):
                    sl = pl.ds(jj * LANES, LANES)
                    bbuf[gs, ei, sl] = jnp.maximum(
                        abuf[gs, ei, sl] + bbuf[gs, ei, sl], 0.0)
                return 0
            lax.fori_loop(0, ch, e_body, 0)

            pltpu.sync_copy(bbuf.at[gs], agg_sh.at[idx_v.at[rb]],
                            add=True)
            for kk in range(ch // LANES):
                plsc.addupdate_scatter(
                    deg_v, [idx_v[rb, pl.ds(kk * LANES, LANES)]],
                    ones16)
            rem = ch - (ch // LANES) * LANES
            if rem:
                tail = lax.iota(jnp.int32, LANES) >= (LANES - rem)
                plsc.addupdate_scatter(
                    deg_v, [idx_v[rb, pl.ds(ch - LANES, LANES)]],
                    ones16, mask=tail)

        sup_fetch(0, 0)
        gather_issue(0, K, 0)

        def sup_pair(q, _):
            for u in (0, 1):
                for j in range(K):
                    chunk_run(2 * q + u, u, j, True, True)
            return 0
        lax.fori_loop(0, (nsup - 1) // 2, sup_pair, 0)
        for j in range(K):
            chunk_run(nsup - 1, 0, j, j < K - 1, False)

        plsc.subcore_barrier()
        for rnd in range(nround):
            bi = si + NS * rnd

            @pl.when(bi < nblk)
            def _():
                pltpu.sync_copy(agg_sh.at[pl.ds(bi * br, br)],
                                agg_out.at[pl.ds(ci * n + bi * br, br)])
        pltpu.sync_copy(deg_v, deg_out.at[pl.ds(wid * n, n)])

    return k(a, b, rowm, colm)


def _update_body(x_ref, g0, g1, dgr, w2, b2r, w3a, w3b, b3r, w4, b4r,
                 gm, bt, o_ref):
    xb = x_ref[...]
    agg = jnp.dot(g0[...] + g1[...], w2[...],
                  preferred_element_type=jnp.float32)
    deg = jnp.sum(dgr[...], axis=1, keepdims=True)
    agg = agg + deg * b2r[...]
    h = jnp.maximum(
        jnp.dot(xb, w3a[...], preferred_element_type=jnp.float32)
        + jnp.dot(agg, w3b[...], preferred_element_type=jnp.float32)
        + b3r[...], 0.0)
    u = jnp.dot(h, w4[...], preferred_element_type=jnp.float32) + b4r[...]
    mu = jnp.mean(u, axis=-1, keepdims=True)
    var = jnp.mean((u - mu) ** 2, axis=-1, keepdims=True)
    ln = (u - mu) * lax.rsqrt(var + 1e-5) * gm[...] + bt[...]
    o_ref[...] = ln + xb


def _update(x, aggp, degt, w2, b2, w3a, w3b, b3, w4, b4, gamma, beta):
    n, d = x.shape
    nw = degt.shape[1]
    bm = 2000
    nb = n // bm
    full = lambda i: (0, 0)
    blk = lambda i: (i, 0)
    return pl.pallas_call(
        _update_body,
        grid=(nb,),
        in_specs=[
            pl.BlockSpec((bm, d), blk),
            pl.BlockSpec((bm, d), blk),
            pl.BlockSpec((bm, d), lambda i: (i + nb, 0)),
            pl.BlockSpec((bm, nw), blk),
            pl.BlockSpec((d, d), full),
            pl.BlockSpec((1, d), full),
            pl.BlockSpec((d, d), full),
            pl.BlockSpec((d, d), full),
            pl.BlockSpec((1, d), full),
            pl.BlockSpec((d, d), full),
            pl.BlockSpec((1, d), full),
            pl.BlockSpec((1, d), full),
            pl.BlockSpec((1, d), full),
        ],
        out_specs=pl.BlockSpec((bm, d), blk),
        out_shape=jax.ShapeDtypeStruct((n, d), jnp.float32),
    )(x, aggp, aggp, degt, w2, b2.reshape(1, d), w3a, w3b, b3.reshape(1, d),
      w4, b4.reshape(1, d), gamma.reshape(1, d), beta.reshape(1, d))


def kernel(x, edge_index, W1, b1, W2, b2, W3, b3, W4, b4, gamma, beta):
    n, d = x.shape
    e = edge_index.shape[1]
    nw = NC * NS
    ch = 40
    kk = 10
    nchunk = e // (nw * ch)
    nsup = nchunk // kk
    assert nsup * kk == nchunk and nsup % 2 == 1
    rowm = edge_index[0].reshape(nw * nsup, kk, ch)
    colm = edge_index[1].reshape(nw * nsup, kk, ch)
    a, bnode = _premix(x, W1[:d], W1[d:], b1)
    aggp, degp = _edge_agg(a, bnode, rowm, colm)
    degt = degp.reshape(nw, n).T
    return _update(x, aggp, degt,
                   W2, b2, W3[:d], W3[d:], b3, W4, b4, gamma, beta)
